# baseline (speedup 1.0000x reference)
import jax
import jax.numpy as jnp
from jax.experimental import pallas as pl
from jax.experimental.pallas import tpu as pltpu

_KB = 256
_K3 = 512


def _body(state_ref, w1_ref, b1_ref, w2_ref, b2_ref, w3_ref, b3_ref,
          v1_ref, vb1_ref, v2_ref, vb2_ref, v3_ref, vb3_ref, g_ref,
          probs_ref, value_ref, act_ref, alp_ref,
          sb, ha, hb, acc, lg):
    B, S = state_ref.shape
    A = b3_ref.shape[1]
    nk = S // _KB
    n3 = S // _K3
    o2 = nk
    o3 = 2 * nk
    o4 = o3 + n3
    o5 = o4 + nk
    o6 = o5 + nk

    i = pl.program_id(0)

    @pl.when(i == 0)
    def _cast_state():
        sb[...] = state_ref[...].astype(jnp.bfloat16)

    def _layer(k, lhs, w_ref, b_ref, out, kb):
        part = jnp.dot(lhs[:, pl.ds(k * kb, kb)], w_ref[...],
                       preferred_element_type=jnp.float32)

        @pl.when(k == 0)
        def _():
            acc[...] = part

        @pl.when(k > 0)
        def _():
            acc[...] = acc[...] + part

        @pl.when(k == S // kb - 1)
        def _():
            out[...] = jnp.tanh(acc[...] + b_ref[...]).astype(jnp.bfloat16)

    @pl.when(i < o2)
    def _l0():
        _layer(i, sb, w1_ref, b1_ref, ha, _KB)

    @pl.when((i >= o2) & (i < o3))
    def _l1():
        _layer(i - o2, ha, w2_ref, b2_ref, hb, _KB)

    @pl.when((i >= o3) & (i < o4))
    def _l2():
        k = i - o3
        part = jnp.dot(hb[:, pl.ds(k * _K3, _K3)], w3_ref[...],
                       preferred_element_type=jnp.float32)

        @pl.when(k == 0)
        def _():
            lg[...] = part

        @pl.when(k > 0)
        def _():
            lg[...] = lg[...] + part

    @pl.when((i >= o4) & (i < o5))
    def _l3():
        _layer(i - o4, sb, v1_ref, vb1_ref, ha, _KB)

    @pl.when((i >= o5) & (i < o6))
    def _l4():
        _layer(i - o5, ha, v2_ref, vb2_ref, hb, _KB)

    @pl.when(i == o6)
    def _fin():
        v3row = v3_ref[...].astype(jnp.bfloat16).astype(jnp.float32)
        hv = hb[...].astype(jnp.float32)
        value_ref[...] = (jnp.sum(hv * v3row, axis=-1, keepdims=True)
                          + vb3_ref[...])
        logits = lg[...] + b3_ref[...]
        m = jnp.max(logits, axis=-1, keepdims=True)
        e = jnp.exp(logits - m)
        p = e / jnp.sum(e, axis=-1, keepdims=True)
        probs_ref[...] = p
        lp = jnp.log(p + 1e-20)
        y = lp + g_ref[...]
        ym = jnp.max(y, axis=-1, keepdims=True)
        cols = jax.lax.broadcasted_iota(jnp.int32, (B, A), 1)
        idx = jnp.min(jnp.where(y == ym, cols, A), axis=-1, keepdims=True)
        act_ref[...] = idx
        alp_ref[...] = jnp.sum(jnp.where(cols == idx, lp, 0.0),
                               axis=-1, keepdims=True)


def kernel(state, W1, b1, W2, b2, W3, b3, V1, vb1, V2, vb2, V3, vb3):
    B, S = state.shape
    H = W1.shape[1]
    A = W3.shape[1]
    nk = S // _KB
    n3 = S // _K3
    o2, o3 = nk, 2 * nk
    o4 = o3 + n3
    o5 = o4 + nk
    o6 = o5 + nk
    steps = o6 + 1

    with jax.ensure_compile_time_eval():
        g = jax.random.gumbel(jax.random.key(42), (B, A), jnp.float32)

    in_specs = [
        pl.BlockSpec((B, S), lambda i: (0, 0)),
        pl.BlockSpec((_KB, H), lambda i: (jnp.clip(i, 0, nk - 1), 0)),
        pl.BlockSpec((1, H), lambda i: (0, 0)),
        pl.BlockSpec((_KB, H), lambda i: (jnp.clip(i - o2, 0, nk - 1), 0)),
        pl.BlockSpec((1, H), lambda i: (0, 0)),
        pl.BlockSpec((_K3, A), lambda i: (jnp.clip(i - o3, 0, n3 - 1), 0)),
        pl.BlockSpec((1, A), lambda i: (0, 0)),
        pl.BlockSpec((_KB, H), lambda i: (jnp.clip(i - o4, 0, nk - 1), 0)),
        pl.BlockSpec((1, H), lambda i: (0, 0)),
        pl.BlockSpec((_KB, H), lambda i: (jnp.clip(i - o5, 0, nk - 1), 0)),
        pl.BlockSpec((1, H), lambda i: (0, 0)),
        pl.BlockSpec((1, S), lambda i: (0, 0)),
        pl.BlockSpec((1, 1), lambda i: (0, 0)),
        pl.BlockSpec((B, A), lambda i: (0, 0)),
    ]
    out_specs = [
        pl.BlockSpec((B, A), lambda i: (0, 0)),
        pl.BlockSpec((B, 1), lambda i: (0, 0)),
        pl.BlockSpec((B, 1), lambda i: (0, 0)),
        pl.BlockSpec((B, 1), lambda i: (0, 0)),
    ]
    out_shape = [
        jax.ShapeDtypeStruct((B, A), jnp.float32),
        jax.ShapeDtypeStruct((B, 1), jnp.float32),
        jax.ShapeDtypeStruct((B, 1), jnp.int32),
        jax.ShapeDtypeStruct((B, 1), jnp.float32),
    ]
    scratch_shapes = [
        pltpu.VMEM((B, S), jnp.bfloat16),
        pltpu.VMEM((B, H), jnp.bfloat16),
        pltpu.VMEM((B, H), jnp.bfloat16),
        pltpu.VMEM((B, H), jnp.float32),
        pltpu.VMEM((B, A), jnp.float32),
    ]

    probs, value, act, alp = pl.pallas_call(
        _body,
        grid=(steps,),
        in_specs=in_specs,
        out_specs=out_specs,
        out_shape=out_shape,
        scratch_shapes=scratch_shapes,
    )(state, W1, b1.reshape(1, H), W2, b2.reshape(1, H),
      W3, b3.reshape(1, A), V1, vb1.reshape(1, H), V2, vb2.reshape(1, H),
      V3.reshape(1, S), vb3.reshape(1, 1), g)
    return probs, value, act[:, 0], alp[:, 0]


# speedup vs baseline: 1.2005x; 1.2005x over previous
import jax
import jax.numpy as jnp
from jax.experimental import pallas as pl
from jax.experimental.pallas import tpu as pltpu

_BN = 256
_AB = 256


def _body(state_ref, w1_ref, b1_ref, w2_ref, b2_ref, w3_ref, b3_ref,
          v1_ref, vb1_ref, v2_ref, vb2_ref, v3_ref, vb3_ref, g_ref,
          probs_ref, value_ref, act_ref, alp_ref,
          sb, h1a, h2a, h1v, h2v, lg):
    B, S = state_ref.shape
    A = b3_ref.shape[1]
    H = h1a.shape[1]
    nb = H // _BN
    na = lg.shape[1] // _AB
    o2 = nb
    o3 = 2 * nb
    o4 = o3 + na

    i = pl.program_id(0)

    @pl.when(i == 0)
    def _cast_state():
        sb[...] = state_ref[...].astype(jnp.bfloat16)

    @pl.when(i < o2)
    def _p0():
        j = i
        xa = jnp.dot(sb[...], w1_ref[...], preferred_element_type=jnp.float32)
        h1a[:, pl.ds(j * _BN, _BN)] = jnp.tanh(xa + b1_ref[...]).astype(jnp.bfloat16)
        xv = jnp.dot(sb[...], v1_ref[...], preferred_element_type=jnp.float32)
        h1v[:, pl.ds(j * _BN, _BN)] = jnp.tanh(xv + vb1_ref[...]).astype(jnp.bfloat16)

    @pl.when((i >= o2) & (i < o3))
    def _p1():
        j = i - o2
        xa = jnp.dot(h1a[...], w2_ref[...], preferred_element_type=jnp.float32)
        h2a[:, pl.ds(j * _BN, _BN)] = jnp.tanh(xa + b2_ref[...]).astype(jnp.bfloat16)
        xv = jnp.dot(h1v[...], v2_ref[...], preferred_element_type=jnp.float32)
        h2v[:, pl.ds(j * _BN, _BN)] = jnp.tanh(xv + vb2_ref[...]).astype(jnp.bfloat16)

    @pl.when((i >= o3) & (i < o4))
    def _p2():
        j = i - o3
        lg[:, pl.ds(j * _AB, _AB)] = jnp.dot(
            h2a[...], w3_ref[...], preferred_element_type=jnp.float32)

    @pl.when(i == o4)
    def _fin():
        v3row = v3_ref[...].astype(jnp.bfloat16).astype(jnp.float32)
        hv = h2v[...].astype(jnp.float32)
        value_ref[...] = (jnp.sum(hv * v3row, axis=-1, keepdims=True)
                          + vb3_ref[...])
        logits = lg[:, :A] + b3_ref[...]
        m = jnp.max(logits, axis=-1, keepdims=True)
        e = jnp.exp(logits - m)
        p = e / jnp.sum(e, axis=-1, keepdims=True)
        probs_ref[...] = p
        lp = jnp.log(p + 1e-20)
        y = lp + g_ref[...]
        ym = jnp.max(y, axis=-1, keepdims=True)
        cols = jax.lax.broadcasted_iota(jnp.int32, (B, A), 1)
        idx = jnp.min(jnp.where(y == ym, cols, A), axis=-1, keepdims=True)
        act_ref[...] = idx
        alp_ref[...] = jnp.sum(jnp.where(cols == idx, lp, 0.0),
                               axis=-1, keepdims=True)


def kernel(state, W1, b1, W2, b2, W3, b3, V1, vb1, V2, vb2, V3, vb3):
    B, S = state.shape
    H = W1.shape[1]
    A = W3.shape[1]
    nb = H // _BN
    na = pl.cdiv(A, _AB)
    Ap = na * _AB
    o2, o3 = nb, 2 * nb
    o4 = o3 + na
    steps = o4 + 1

    with jax.ensure_compile_time_eval():
        g = jax.random.gumbel(jax.random.key(42), (B, A), jnp.float32)

    in_specs = [
        pl.BlockSpec((B, S), lambda i: (0, 0)),
        pl.BlockSpec((S, _BN), lambda i: (0, jnp.clip(i, 0, nb - 1))),
        pl.BlockSpec((1, _BN), lambda i: (0, jnp.clip(i, 0, nb - 1))),
        pl.BlockSpec((H, _BN), lambda i: (0, jnp.clip(i - o2, 0, nb - 1))),
        pl.BlockSpec((1, _BN), lambda i: (0, jnp.clip(i - o2, 0, nb - 1))),
        pl.BlockSpec((S, _AB), lambda i: (0, jnp.clip(i - o3, 0, na - 1))),
        pl.BlockSpec((1, A), lambda i: (0, 0)),
        pl.BlockSpec((S, _BN), lambda i: (0, jnp.clip(i, 0, nb - 1))),
        pl.BlockSpec((1, _BN), lambda i: (0, jnp.clip(i, 0, nb - 1))),
        pl.BlockSpec((H, _BN), lambda i: (0, jnp.clip(i - o2, 0, nb - 1))),
        pl.BlockSpec((1, _BN), lambda i: (0, jnp.clip(i - o2, 0, nb - 1))),
        pl.BlockSpec((1, S), lambda i: (0, 0)),
        pl.BlockSpec((1, 1), lambda i: (0, 0)),
        pl.BlockSpec((B, A), lambda i: (0, 0)),
    ]
    out_specs = [
        pl.BlockSpec((B, A), lambda i: (0, 0)),
        pl.BlockSpec((B, 1), lambda i: (0, 0)),
        pl.BlockSpec((B, 1), lambda i: (0, 0)),
        pl.BlockSpec((B, 1), lambda i: (0, 0)),
    ]
    out_shape = [
        jax.ShapeDtypeStruct((B, A), jnp.float32),
        jax.ShapeDtypeStruct((B, 1), jnp.float32),
        jax.ShapeDtypeStruct((B, 1), jnp.int32),
        jax.ShapeDtypeStruct((B, 1), jnp.float32),
    ]
    scratch_shapes = [
        pltpu.VMEM((B, S), jnp.bfloat16),
        pltpu.VMEM((B, H), jnp.bfloat16),
        pltpu.VMEM((B, H), jnp.bfloat16),
        pltpu.VMEM((B, H), jnp.bfloat16),
        pltpu.VMEM((B, H), jnp.bfloat16),
        pltpu.VMEM((B, Ap), jnp.float32),
    ]

    probs, value, act, alp = pl.pallas_call(
        _body,
        grid=(steps,),
        in_specs=in_specs,
        out_specs=out_specs,
        out_shape=out_shape,
        scratch_shapes=scratch_shapes,
    )(state, W1, b1.reshape(1, H), W2, b2.reshape(1, H),
      W3, b3.reshape(1, A), V1, vb1.reshape(1, H), V2, vb2.reshape(1, H),
      V3.reshape(1, S), vb3.reshape(1, 1), g)
    return probs, value, act[:, 0], alp[:, 0]
